# baseline (speedup 1.0000x reference)
import functools

import jax
import jax.numpy as jnp
from jax import lax
from jax.experimental import pallas as pl
from jax.experimental.pallas import tpu as pltpu
from jax.experimental.pallas import tpu_sc as plsc

_EPS = 1e-5
_NC, _NS = 2, 16
_NW = _NC * _NS
_CH = 128
_C2 = 64


def _cdiv(a, b):
    return (a + b - 1) // b


def _sc_gather_rows(table, idx_pad, H):
    n_pad = idx_pad.shape[0]
    k_per_w = n_pad // (_NW * _CH)
    mesh = plsc.VectorSubcoreMesh(core_axis_name="c", subcore_axis_name="s")

    @functools.partial(
        pl.kernel,
        out_type=jax.ShapeDtypeStruct((n_pad, H), jnp.float32),
        mesh=mesh,
        scratch_types=[
            pltpu.VMEM((_CH,), jnp.int32),
            pltpu.VMEM((_CH, H), jnp.float32),
            pltpu.SemaphoreType.DMA,
        ],
    )
    def k(tab_hbm, idx_hbm, out_hbm, idx, rows, sem):
        cid = lax.axis_index("c")
        sid = lax.axis_index("s")
        wid = sid * _NC + cid
        base = wid * (k_per_w * _CH)

        def body(j, carry):
            off = base + j * _CH
            pltpu.sync_copy(idx_hbm.at[pl.ds(off, _CH)], idx)
            pltpu.async_copy(tab_hbm.at[idx], rows, sem).wait()
            pltpu.sync_copy(rows, out_hbm.at[pl.ds(off, _CH)])
            return carry

        lax.fori_loop(0, k_per_w, body, 0)

    return k(table, idx_pad)


def _sc_segment_sum(hp, srcf, dstf, n_half):
    H = hp.shape[1]
    HP = hp.shape[0]
    K = srcf.shape[1] // _C2
    rph = HP // _NS
    rpa = n_half // _NS
    mesh = plsc.VectorSubcoreMesh(core_axis_name="c", subcore_axis_name="s")

    @functools.partial(
        pl.kernel,
        out_type=jax.ShapeDtypeStruct((2 * n_half, H), jnp.float32),
        mesh=mesh,
        scratch_types=[
            [pltpu.VMEM((_C2,), jnp.int32) for _ in range(2)],
            [pltpu.VMEM((_C2,), jnp.int32) for _ in range(2)],
            pltpu.VMEM((_C2, H), jnp.float32),
            pltpu.VMEM_SHARED((HP, H), jnp.float32),
            pltpu.VMEM_SHARED((n_half, H), jnp.float32),
            pltpu.SemaphoreType.DMA,
            pltpu.SemaphoreType.DMA,
            pltpu.SemaphoreType.DMA,
        ],
    )
    def k(hp_hbm, src_hbm, dst_hbm, out_hbm, sring, dring, rows, hsh, acc,
          gsem, isem, dsem):
        cid = lax.axis_index("c")
        sid = lax.axis_index("s")
        rr = cid * _NS + sid

        def zrow(i, carry):
            for t in range(H // 16):
                rows[i, pl.ds(t * 16, 16)] = jnp.zeros((16,), jnp.float32)
            return carry

        lax.fori_loop(0, _C2, zrow, 0)

        def zcp(i, carry):
            pltpu.sync_copy(rows, acc.at[pl.ds(sid * rpa + i * _C2, _C2)])
            return carry

        lax.fori_loop(0, rpa // _C2, zcp, 0)
        pltpu.sync_copy(hp_hbm.at[pl.ds(sid * rph, rph)],
                        hsh.at[pl.ds(sid * rph, rph)])
        for p in range(2):
            pltpu.async_copy(
                src_hbm.at[sid, pl.ds(p * _C2, _C2)], sring[p], isem)
            pltpu.async_copy(
                dst_hbm.at[rr, pl.ds(p * _C2, _C2)], dring[p], dsem)
        plsc.subcore_barrier()

        def body(g, carry):
            for b0 in range(2):
                ch = g * 2 + b0
                pltpu.make_async_copy(
                    src_hbm.at[sid, pl.ds(0, _C2)], sring[b0], isem).wait()
                pltpu.make_async_copy(
                    dst_hbm.at[rr, pl.ds(0, _C2)], dring[b0], dsem).wait()
                pltpu.async_copy(hsh.at[sring[b0]], rows, gsem)
                pltpu.make_async_copy(hsh.at[sring[b0]], rows, gsem).wait()
                nxt = lax.rem(ch + 2, K)
                pltpu.async_copy(
                    src_hbm.at[sid, pl.ds(nxt * _C2, _C2)], sring[b0], isem)
                pltpu.sync_copy(rows, acc.at[dring[b0]], add=True)
                pltpu.async_copy(
                    dst_hbm.at[rr, pl.ds(nxt * _C2, _C2)], dring[b0], dsem)
            return carry

        lax.fori_loop(0, K // 2, body, 0)
        for p in range(2):
            pltpu.make_async_copy(
                src_hbm.at[sid, pl.ds(0, _C2)], sring[p], isem).wait()
            pltpu.make_async_copy(
                dst_hbm.at[rr, pl.ds(0, _C2)], dring[p], dsem).wait()
        plsc.subcore_barrier()
        pltpu.sync_copy(
            acc.at[pl.ds(sid * rpa, rpa)],
            out_hbm.at[pl.ds(cid * n_half + sid * rpa, rpa)],
        )

    return k(hp, srcf, dstf)


def _sc_degree(dst3, n_pad, H):
    K = dst3.shape[1]
    rpt = n_pad // _NS
    mesh = plsc.VectorSubcoreMesh(core_axis_name="c", subcore_axis_name="s")

    @functools.partial(
        pl.kernel,
        out_type=jax.ShapeDtypeStruct((2 * n_pad, H), jnp.float32),
        mesh=mesh,
        scratch_types=[
            pltpu.VMEM((K, _CH), jnp.int32),
            pltpu.VMEM((_CH, H), jnp.float32),
            pltpu.VMEM_SHARED((n_pad, H), jnp.float32),
        ],
    )
    def k(dst_hbm, out_hbm, didx, ones, acc):
        cid = lax.axis_index("c")
        sid = lax.axis_index("s")
        wid = sid * _NC + cid

        def zrow(i, carry):
            for t in range(H // 16):
                ones[i, pl.ds(t * 16, 16)] = jnp.zeros((16,), jnp.float32)
            return carry

        lax.fori_loop(0, _CH, zrow, 0)

        def zcp(i, carry):
            pltpu.sync_copy(ones, acc.at[pl.ds(sid * rpt + i * _CH, _CH)])
            return carry

        lax.fori_loop(0, rpt // _CH, zcp, 0)

        def orow(i, carry):
            for t in range(H // 16):
                ones[i, pl.ds(t * 16, 16)] = jnp.ones((16,), jnp.float32)
            return carry

        lax.fori_loop(0, _CH, orow, 0)
        pltpu.sync_copy(dst_hbm.at[wid], didx)
        plsc.subcore_barrier()

        def body(j, carry):
            pltpu.sync_copy(ones, acc.at[didx.at[j]], add=True)
            return carry

        lax.fori_loop(0, K, body, 0)
        plsc.subcore_barrier()
        pltpu.sync_copy(
            acc.at[pl.ds(sid * rpt, rpt)],
            out_hbm.at[pl.ds(cid * n_pad + sid * rpt, rpt)],
        )

    return k(dst3)


def _tc_pre(part, degp, h, wl, wr, bl, N, B, H):
    grid = N // B

    hb = grid // 2

    def body(p_ref, d_ref, h_ref, wl_ref, wr_ref, b_ref, z_ref, s_ref, s2_ref):
        i = pl.program_id(0)
        deg = d_ref[0, :, 0:1] + d_ref[1, :, 0:1]
        rdeg = 1.0 / jnp.maximum(deg, 1.0)
        agg = p_ref[0] * rdeg
        z = (jnp.dot(agg, wl_ref[...], preferred_element_type=jnp.float32)
             + jnp.dot(h_ref[...], wr_ref[...], preferred_element_type=jnp.float32)
             + b_ref[...])
        z_ref[...] = z

        @pl.when(i == 0)
        def _():
            s_ref[...] = jnp.zeros_like(s_ref)
            s2_ref[...] = jnp.zeros_like(s2_ref)

        s_ref[...] += jnp.sum(z, axis=0, keepdims=True)
        s2_ref[...] += jnp.sum(z * z, axis=0, keepdims=True)

    return pl.pallas_call(
        body,
        grid=(grid,),
        in_specs=[
            pl.BlockSpec((1, B, H), lambda i: (i // hb, i % hb, 0)),
            pl.BlockSpec((2, B, H), lambda i: (0, i, 0)),
            pl.BlockSpec((B, H), lambda i: (i, 0)),
            pl.BlockSpec((H, H), lambda i: (0, 0)),
            pl.BlockSpec((H, H), lambda i: (0, 0)),
            pl.BlockSpec((1, H), lambda i: (0, 0)),
        ],
        out_specs=[
            pl.BlockSpec((B, H), lambda i: (i, 0)),
            pl.BlockSpec((1, H), lambda i: (0, 0)),
            pl.BlockSpec((1, H), lambda i: (0, 0)),
        ],
        out_shape=[
            jax.ShapeDtypeStruct((N, H), jnp.float32),
            jax.ShapeDtypeStruct((1, H), jnp.float32),
            jax.ShapeDtypeStruct((1, H), jnp.float32),
        ],
    )(part, degp, h, wl, wr, bl)


def _tc_post(z, s, s2, g, be, N, B, H):
    grid = N // B

    def body(z_ref, s_ref, s2_ref, g_ref, b_ref, h_ref):
        m = s_ref[...] * (1.0 / N)
        v = s2_ref[...] * (1.0 / N) - m * m
        inv = lax.rsqrt(v + _EPS)
        h_ref[...] = jnp.maximum(
            (z_ref[...] - m) * inv * g_ref[...] + b_ref[...], 0.0)

    return pl.pallas_call(
        body,
        grid=(grid,),
        in_specs=[
            pl.BlockSpec((B, H), lambda i: (i, 0)),
            pl.BlockSpec((1, H), lambda i: (0, 0)),
            pl.BlockSpec((1, H), lambda i: (0, 0)),
            pl.BlockSpec((1, H), lambda i: (0, 0)),
            pl.BlockSpec((1, H), lambda i: (0, 0)),
        ],
        out_specs=pl.BlockSpec((B, H), lambda i: (i, 0)),
        out_shape=jax.ShapeDtypeStruct((N, H), jnp.float32),
    )(z, s, s2, g, be)


def _tc_post_pool(z, s, s2, g, be, batch3, N, B, H, G):
    grid = N // B

    def body(z_ref, s_ref, s2_ref, g_ref, b_ref, bt_ref, out_ref, acc, cacc):
        i = pl.program_id(0)
        m = s_ref[...] * (1.0 / N)
        v = s2_ref[...] * (1.0 / N) - m * m
        inv = lax.rsqrt(v + _EPS)
        h = jnp.maximum((z_ref[...] - m) * inv * g_ref[...] + b_ref[...], 0.0)
        gids = lax.broadcasted_iota(jnp.int32, (G, B), 0)
        oh = (bt_ref[0] == gids).astype(jnp.float32)

        @pl.when(i == 0)
        def _():
            acc[...] = jnp.zeros_like(acc)
            cacc[...] = jnp.zeros_like(cacc)

        acc[...] += jnp.dot(oh, h, preferred_element_type=jnp.float32)
        cacc[...] += jnp.sum(oh, axis=1, keepdims=True)

        @pl.when(i == grid - 1)
        def _():
            out_ref[...] = acc[...] / jnp.maximum(cacc[...], 1.0)

    return pl.pallas_call(
        body,
        grid=(grid,),
        in_specs=[
            pl.BlockSpec((B, H), lambda i: (i, 0)),
            pl.BlockSpec((1, H), lambda i: (0, 0)),
            pl.BlockSpec((1, H), lambda i: (0, 0)),
            pl.BlockSpec((1, H), lambda i: (0, 0)),
            pl.BlockSpec((1, H), lambda i: (0, 0)),
            pl.BlockSpec((1, 1, B), lambda i: (i, 0, 0)),
        ],
        out_specs=pl.BlockSpec((G, H), lambda i: (0, 0)),
        out_shape=jax.ShapeDtypeStruct((G, H), jnp.float32),
        scratch_shapes=[
            pltpu.VMEM((G, H), jnp.float32),
            pltpu.VMEM((G, 1), jnp.float32),
        ],
    )(z, s, s2, g, be, batch3)


def kernel(x, edge_index, batch, u_index, emb, Wl, Wr, b, gamma, beta,
           mlp_W1, mlp_b1, mlp_g, mlp_be, mlp_W2, mlp_b2):
    N = x.shape[0]
    E = edge_index.shape[1]
    H = emb.shape[1]
    L = Wl.shape[0]
    G = 16
    B = 1000

    n_pad = _NS * _CH * _cdiv(N + 1, _NS * _CH)
    half = N // 2
    n_half = 1024 * _cdiv(half + 57, 1024)
    HP = _NS * 8 * _cdiv(N, _NS * 8)
    trash = half + 56
    x_pad = _NW * _CH * _cdiv(N, _NW * _CH)
    ept = _cdiv(E, _NS)
    K = 2 * _cdiv(_cdiv(ept, _C2), 2)
    kd = _cdiv(_cdiv(E, _NW * _CH), 4) * 4

    src = edge_index[0]
    dst = edge_index[1]
    src2 = jnp.concatenate([src, jnp.zeros((_NS * ept - E,), src.dtype)])
    src2 = src2.reshape(_NS, ept)
    srcf = jnp.pad(src2, ((0, 0), (0, K * _C2 - ept)))
    dst2 = jnp.concatenate([dst, jnp.full((_NS * ept - E,), N, dst.dtype)])
    dst2 = dst2.reshape(_NS, ept)
    dst2 = jnp.pad(dst2, ((0, 0), (0, K * _C2 - ept)), constant_values=N)
    loc0 = jnp.where(dst2 < half, dst2, trash)
    loc1 = jnp.where(dst2 >= half, dst2 - half, trash)
    dstf = jnp.stack([loc0, loc1]).reshape(2 * _NS, K * _C2)

    e_padd = _NW * kd * _CH
    dst3 = jnp.concatenate(
        [dst, jnp.full((e_padd - E,), N, dst.dtype)]).reshape(_NW, kd, _CH)
    xp = jnp.concatenate([x, jnp.zeros((x_pad - N,), x.dtype)])

    h = _sc_gather_rows(emb, xp, H)[:N]
    degp = _sc_degree(dst3, n_pad, H).reshape(2, n_pad, H)
    batch3 = batch.reshape(N // B, 1, B)
    hzero = jnp.zeros((HP - N, H), jnp.float32)

    pooled = None
    for l in range(L):
        hp = jnp.concatenate([h, hzero])
        part = _sc_segment_sum(hp, srcf, dstf, n_half).reshape(2, n_half, H)
        z, s, s2 = _tc_pre(part, degp, h, Wl[l], Wr[l], b[l].reshape(1, H),
                           N, B, H)
        gl = gamma[l].reshape(1, H)
        bl = beta[l].reshape(1, H)
        if l < L - 1:
            h = _tc_post(z, s, s2, gl, bl, N, B, H)
        else:
            pooled = _tc_post_pool(z, s, s2, gl, bl, batch3, N, B, H, G)
    return pooled


# speedup vs baseline: 1.5437x; 1.5437x over previous
import functools

import jax
import jax.numpy as jnp
from jax import lax
from jax.experimental import pallas as pl
from jax.experimental.pallas import tpu as pltpu
from jax.experimental.pallas import tpu_sc as plsc

_EPS = 1e-5
_NC, _NS = 2, 16
_NW = _NC * _NS
_CH = 128
_C2 = 64


def _cdiv(a, b):
    return (a + b - 1) // b


def _sc_gather_rows(table, idx_pad, H):
    n_pad = idx_pad.shape[0]
    k_per_w = n_pad // (_NW * _CH)
    mesh = plsc.VectorSubcoreMesh(core_axis_name="c", subcore_axis_name="s")

    @functools.partial(
        pl.kernel,
        out_type=jax.ShapeDtypeStruct((n_pad, H), jnp.float32),
        mesh=mesh,
        scratch_types=[
            pltpu.VMEM((_CH,), jnp.int32),
            pltpu.VMEM((_CH, H), jnp.float32),
            pltpu.SemaphoreType.DMA,
        ],
    )
    def k(tab_hbm, idx_hbm, out_hbm, idx, rows, sem):
        cid = lax.axis_index("c")
        sid = lax.axis_index("s")
        wid = sid * _NC + cid
        base = wid * (k_per_w * _CH)

        def body(j, carry):
            off = base + j * _CH
            pltpu.sync_copy(idx_hbm.at[pl.ds(off, _CH)], idx)
            pltpu.async_copy(tab_hbm.at[idx], rows, sem).wait()
            pltpu.sync_copy(rows, out_hbm.at[pl.ds(off, _CH)])
            return carry

        lax.fori_loop(0, k_per_w, body, 0)

    return k(table, idx_pad)


def _sc_segment_sum(hp, srcs, dstl, scal, n_half):
    H = hp.shape[1]
    HP = hp.shape[0]
    rph = HP // _NS
    rpa = n_half // _NS
    mesh = plsc.VectorSubcoreMesh(core_axis_name="c", subcore_axis_name="s")

    @functools.partial(
        pl.kernel,
        out_type=jax.ShapeDtypeStruct((2 * n_half, H), jnp.float32),
        mesh=mesh,
        scratch_types=[
            [pltpu.VMEM((_C2,), jnp.int32) for _ in range(2)],
            [pltpu.VMEM((_C2,), jnp.int32) for _ in range(2)],
            pltpu.VMEM((_C2, H), jnp.float32),
            pltpu.VMEM_SHARED((HP, H), jnp.float32),
            pltpu.VMEM_SHARED((n_half, H), jnp.float32),
            pltpu.VMEM((16,), jnp.int32),
            pltpu.SemaphoreType.DMA,
            pltpu.SemaphoreType.DMA,
            pltpu.SemaphoreType.DMA,
        ],
    )
    def k(hp_hbm, src_hbm, dst_hbm, scal_hbm, out_hbm, sring, dring, rows,
          hsh, acc, sm, gsem, isem, dsem):
        cid = lax.axis_index("c")
        sid = lax.axis_index("s")
        pltpu.sync_copy(scal_hbm, sm)
        sv = sm[...]
        cpt = jnp.where(cid == 0, sv[0], sv[1])
        start = jnp.where(cid == 0, sv[2], sv[3])
        base = pl.multiple_of(start + sid * cpt * _C2, _C2)

        def zrow(i, carry):
            for t in range(H // 16):
                rows[i, pl.ds(t * 16, 16)] = jnp.zeros((16,), jnp.float32)
            return carry

        lax.fori_loop(0, _C2, zrow, 0)

        def zcp(i, carry):
            pltpu.sync_copy(rows, acc.at[pl.ds(sid * rpa + i * _C2, _C2)])
            return carry

        lax.fori_loop(0, rpa // _C2, zcp, 0)
        pltpu.sync_copy(hp_hbm.at[pl.ds(sid * rph, rph)],
                        hsh.at[pl.ds(sid * rph, rph)])
        for p in range(2):
            pltpu.async_copy(
                src_hbm.at[pl.ds(pl.multiple_of(base + p * _C2, _C2), _C2)], sring[p], isem)
            pltpu.async_copy(
                dst_hbm.at[cid, pl.ds(pl.multiple_of(base + p * _C2, _C2), _C2)], dring[p], dsem)
        plsc.subcore_barrier()

        def body(g, carry):
            for b0 in range(2):
                ch = g * 2 + b0
                pltpu.make_async_copy(
                    src_hbm.at[pl.ds(base, _C2)], sring[b0], isem).wait()
                pltpu.make_async_copy(
                    dst_hbm.at[cid, pl.ds(base, _C2)], dring[b0],
                    dsem).wait()
                pltpu.async_copy(hsh.at[sring[b0]], rows, gsem)
                pltpu.make_async_copy(hsh.at[sring[b0]], rows, gsem).wait()
                nxt = lax.rem(ch + 2, cpt)
                pltpu.async_copy(
                    src_hbm.at[pl.ds(pl.multiple_of(base + nxt * _C2, _C2), _C2)], sring[b0], isem)
                pltpu.sync_copy(rows, acc.at[dring[b0]], add=True)
                pltpu.async_copy(
                    dst_hbm.at[cid, pl.ds(pl.multiple_of(base + nxt * _C2, _C2), _C2)],
                    dring[b0], dsem)
            return carry

        lax.fori_loop(0, cpt // 2, body, 0)
        for p in range(2):
            pltpu.make_async_copy(
                src_hbm.at[pl.ds(base, _C2)], sring[p], isem).wait()
            pltpu.make_async_copy(
                dst_hbm.at[cid, pl.ds(base, _C2)], dring[p], dsem).wait()
        plsc.subcore_barrier()
        pltpu.sync_copy(
            acc.at[pl.ds(sid * rpa, rpa)],
            out_hbm.at[pl.ds(cid * n_half + sid * rpa, rpa)],
        )

    return k(hp, srcs, dstl, scal)


def _sc_degree(dst3, n_pad, H):
    K = dst3.shape[1]
    rpt = n_pad // _NS
    mesh = plsc.VectorSubcoreMesh(core_axis_name="c", subcore_axis_name="s")

    @functools.partial(
        pl.kernel,
        out_type=jax.ShapeDtypeStruct((2 * n_pad, H), jnp.float32),
        mesh=mesh,
        scratch_types=[
            pltpu.VMEM((K, _CH), jnp.int32),
            pltpu.VMEM((_CH, H), jnp.float32),
            pltpu.VMEM_SHARED((n_pad, H), jnp.float32),
        ],
    )
    def k(dst_hbm, out_hbm, didx, ones, acc):
        cid = lax.axis_index("c")
        sid = lax.axis_index("s")
        wid = sid * _NC + cid

        def zrow(i, carry):
            for t in range(H // 16):
                ones[i, pl.ds(t * 16, 16)] = jnp.zeros((16,), jnp.float32)
            return carry

        lax.fori_loop(0, _CH, zrow, 0)

        def zcp(i, carry):
            pltpu.sync_copy(ones, acc.at[pl.ds(sid * rpt + i * _CH, _CH)])
            return carry

        lax.fori_loop(0, rpt // _CH, zcp, 0)

        def orow(i, carry):
            for t in range(H // 16):
                ones[i, pl.ds(t * 16, 16)] = jnp.ones((16,), jnp.float32)
            return carry

        lax.fori_loop(0, _CH, orow, 0)
        pltpu.sync_copy(dst_hbm.at[wid], didx)
        plsc.subcore_barrier()

        def body(j, carry):
            pltpu.sync_copy(ones, acc.at[didx.at[j]], add=True)
            return carry

        lax.fori_loop(0, K, body, 0)
        plsc.subcore_barrier()
        pltpu.sync_copy(
            acc.at[pl.ds(sid * rpt, rpt)],
            out_hbm.at[pl.ds(cid * n_pad + sid * rpt, rpt)],
        )

    return k(dst3)


def _tc_pre(part, degp, h, wl, wr, bl, N, B, H):
    grid = N // B

    hb = grid // 2

    def body(p_ref, d_ref, h_ref, wl_ref, wr_ref, b_ref, z_ref, s_ref, s2_ref):
        i = pl.program_id(0)
        deg = d_ref[0, :, 0:1] + d_ref[1, :, 0:1]
        rdeg = 1.0 / jnp.maximum(deg, 1.0)
        agg = p_ref[0] * rdeg
        z = (jnp.dot(agg, wl_ref[...], preferred_element_type=jnp.float32)
             + jnp.dot(h_ref[...], wr_ref[...], preferred_element_type=jnp.float32)
             + b_ref[...])
        z_ref[...] = z

        @pl.when(i == 0)
        def _():
            s_ref[...] = jnp.zeros_like(s_ref)
            s2_ref[...] = jnp.zeros_like(s2_ref)

        s_ref[...] += jnp.sum(z, axis=0, keepdims=True)
        s2_ref[...] += jnp.sum(z * z, axis=0, keepdims=True)

    return pl.pallas_call(
        body,
        grid=(grid,),
        in_specs=[
            pl.BlockSpec((1, B, H), lambda i: (i // hb, i % hb, 0)),
            pl.BlockSpec((2, B, H), lambda i: (0, i, 0)),
            pl.BlockSpec((B, H), lambda i: (i, 0)),
            pl.BlockSpec((H, H), lambda i: (0, 0)),
            pl.BlockSpec((H, H), lambda i: (0, 0)),
            pl.BlockSpec((1, H), lambda i: (0, 0)),
        ],
        out_specs=[
            pl.BlockSpec((B, H), lambda i: (i, 0)),
            pl.BlockSpec((1, H), lambda i: (0, 0)),
            pl.BlockSpec((1, H), lambda i: (0, 0)),
        ],
        out_shape=[
            jax.ShapeDtypeStruct((N, H), jnp.float32),
            jax.ShapeDtypeStruct((1, H), jnp.float32),
            jax.ShapeDtypeStruct((1, H), jnp.float32),
        ],
    )(part, degp, h, wl, wr, bl)


def _tc_post(z, s, s2, g, be, N, B, H):
    grid = N // B

    def body(z_ref, s_ref, s2_ref, g_ref, b_ref, h_ref):
        m = s_ref[...] * (1.0 / N)
        v = s2_ref[...] * (1.0 / N) - m * m
        inv = lax.rsqrt(v + _EPS)
        h_ref[...] = jnp.maximum(
            (z_ref[...] - m) * inv * g_ref[...] + b_ref[...], 0.0)

    return pl.pallas_call(
        body,
        grid=(grid,),
        in_specs=[
            pl.BlockSpec((B, H), lambda i: (i, 0)),
            pl.BlockSpec((1, H), lambda i: (0, 0)),
            pl.BlockSpec((1, H), lambda i: (0, 0)),
            pl.BlockSpec((1, H), lambda i: (0, 0)),
            pl.BlockSpec((1, H), lambda i: (0, 0)),
        ],
        out_specs=pl.BlockSpec((B, H), lambda i: (i, 0)),
        out_shape=jax.ShapeDtypeStruct((N, H), jnp.float32),
    )(z, s, s2, g, be)


def _tc_post_pool(z, s, s2, g, be, batch3, N, B, H, G):
    grid = N // B

    def body(z_ref, s_ref, s2_ref, g_ref, b_ref, bt_ref, out_ref, acc, cacc):
        i = pl.program_id(0)
        m = s_ref[...] * (1.0 / N)
        v = s2_ref[...] * (1.0 / N) - m * m
        inv = lax.rsqrt(v + _EPS)
        h = jnp.maximum((z_ref[...] - m) * inv * g_ref[...] + b_ref[...], 0.0)
        gids = lax.broadcasted_iota(jnp.int32, (G, B), 0)
        oh = (bt_ref[0] == gids).astype(jnp.float32)

        @pl.when(i == 0)
        def _():
            acc[...] = jnp.zeros_like(acc)
            cacc[...] = jnp.zeros_like(cacc)

        acc[...] += jnp.dot(oh, h, preferred_element_type=jnp.float32)
        cacc[...] += jnp.sum(oh, axis=1, keepdims=True)

        @pl.when(i == grid - 1)
        def _():
            out_ref[...] = acc[...] / jnp.maximum(cacc[...], 1.0)

    return pl.pallas_call(
        body,
        grid=(grid,),
        in_specs=[
            pl.BlockSpec((B, H), lambda i: (i, 0)),
            pl.BlockSpec((1, H), lambda i: (0, 0)),
            pl.BlockSpec((1, H), lambda i: (0, 0)),
            pl.BlockSpec((1, H), lambda i: (0, 0)),
            pl.BlockSpec((1, H), lambda i: (0, 0)),
            pl.BlockSpec((1, 1, B), lambda i: (i, 0, 0)),
        ],
        out_specs=pl.BlockSpec((G, H), lambda i: (0, 0)),
        out_shape=jax.ShapeDtypeStruct((G, H), jnp.float32),
        scratch_shapes=[
            pltpu.VMEM((G, H), jnp.float32),
            pltpu.VMEM((G, 1), jnp.float32),
        ],
    )(z, s, s2, g, be, batch3)


def kernel(x, edge_index, batch, u_index, emb, Wl, Wr, b, gamma, beta,
           mlp_W1, mlp_b1, mlp_g, mlp_be, mlp_W2, mlp_b2):
    N = x.shape[0]
    E = edge_index.shape[1]
    H = emb.shape[1]
    L = Wl.shape[0]
    G = 16
    B = 1000

    n_pad = _NS * _CH * _cdiv(N + 1, _NS * _CH)
    half = N // 2
    n_half = 1024 * _cdiv(half + 57, 1024)
    HP = _NS * 8 * _cdiv(N, _NS * 8)
    trash = half + 56
    x_pad = _NW * _CH * _cdiv(N, _NW * _CH)
    ept = _cdiv(E, _NS)
    K = 2 * _cdiv(_cdiv(ept, _C2), 2)
    kd = _cdiv(_cdiv(E, _NW * _CH), 4) * 4

    src = edge_index[0]
    dst = edge_index[1]
    key = (dst >= half).astype(jnp.int32)
    perm = jnp.argsort(key, stable=True)
    slack = 2 * _NS * _C2
    srcs = jnp.concatenate(
        [jnp.take(src, perm), jnp.zeros((slack,), src.dtype)])
    dsts = jnp.concatenate(
        [jnp.take(dst, perm), jnp.full((slack,), N, dst.dtype)])
    loc0 = jnp.where(dsts < half, dsts, trash)
    loc1 = jnp.where((dsts >= half) & (dsts < N), dsts - half, trash)
    dstl = jnp.stack([loc0, loc1])
    n0 = E - jnp.sum(key)
    c0 = (n0 + _NS * _C2 - 1) // (_NS * _C2)
    cpt0 = c0 + (c0 & 1)
    s1 = (n0 // _C2) * _C2
    c1 = (E - s1 + _NS * _C2 - 1) // (_NS * _C2)
    cpt1 = c1 + (c1 & 1)
    z32 = jnp.zeros((), jnp.int32)
    scal = jnp.stack([cpt0, cpt1, z32, s1] + [z32] * 12).astype(jnp.int32)

    e_padd = _NW * kd * _CH
    dst3 = jnp.concatenate(
        [dst, jnp.full((e_padd - E,), N, dst.dtype)]).reshape(_NW, kd, _CH)
    xp = jnp.concatenate([x, jnp.zeros((x_pad - N,), x.dtype)])

    h = _sc_gather_rows(emb, xp, H)[:N]
    degp = _sc_degree(dst3, n_pad, H).reshape(2, n_pad, H)
    batch3 = batch.reshape(N // B, 1, B)
    hzero = jnp.zeros((HP - N, H), jnp.float32)

    pooled = None
    for l in range(L):
        hp = jnp.concatenate([h, hzero])
        part = _sc_segment_sum(hp, srcs, dstl, scal, n_half
                               ).reshape(2, n_half, H)
        z, s, s2 = _tc_pre(part, degp, h, Wl[l], Wr[l], b[l].reshape(1, H),
                           N, B, H)
        gl = gamma[l].reshape(1, H)
        bl = beta[l].reshape(1, H)
        if l < L - 1:
            h = _tc_post(z, s, s2, gl, bl, N, B, H)
        else:
            pooled = _tc_post_pool(z, s, s2, gl, bl, batch3, N, B, H, G)
    return pooled
